# Initial kernel scaffold; baseline (speedup 1.0000x reference)
import jax
import jax.numpy as jnp
from jax.experimental import pallas as pl


def kernel(xyz, data, h, w):
    raise NotImplementedError("write your pallas kernel here")


# speedup vs baseline: 2.9038x; 2.9038x over previous
import functools

import jax
import jax.numpy as jnp
from jax import lax
from jax.experimental import pallas as pl
from jax.experimental.pallas import tpu as pltpu
from jax.experimental.pallas import tpu_sc as plsc

P = 32768
BS = 4
CH = 65
G = 65536
L = 16
NS = 16
PT = P // NS
CK = 2048
NPASS = 2 * CH


def _splat_body(pxyz, data65, acc, idxs, wks, pbuf, idxb, wkb, dbuf, accb):
    c = lax.axis_index("c")
    s = lax.axis_index("s")

    for bl in range(2):
        b = 2 * c + bl
        base = s * PT
        pltpu.sync_copy(pxyz.at[b, :, pl.ds(base, PT)], pbuf)

        def p1_group(g, _):
            g16 = g * L
            px = pbuf[0, pl.ds(g16, L)]
            py = pbuf[1, pl.ds(g16, L)]
            iz = pbuf[2, pl.ds(g16, L)]
            pxc = jnp.minimum(jnp.maximum(px, 0.0), 255.0)
            pyc = jnp.minimum(jnp.maximum(py, 0.0), 255.0)
            x0 = jnp.minimum(pxc.astype(jnp.int32), 254)
            y0 = jnp.minimum(pyc.astype(jnp.int32), 254)
            fx = pxc - x0.astype(jnp.float32)
            fy = pyc - y0.astype(jnp.float32)
            gx = 1.0 - fx
            gy = 1.0 - fy
            idxb[pl.ds(g16, L)] = y0 * 256 + x0
            wkb[0, pl.ds(g16, L)] = gx * gy * iz
            wkb[1, pl.ds(g16, L)] = fx * gy * iz
            wkb[2, pl.ds(g16, L)] = gx * fy * iz
            wkb[3, pl.ds(g16, L)] = fx * fy * iz
            return 0

        lax.fori_loop(0, PT // L, p1_group, 0)
        pltpu.sync_copy(idxb, idxs.at[b, pl.ds(base, PT)])
        pltpu.sync_copy(wkb, wks.at[b, :, pl.ds(base, PT)])

    plsc.subcore_barrier()

    def do_pass(lin):
        b = 2 * c + lin // CH
        ch = lin % CH

        def zero_block(i, _):
            base = i * (8 * L)
            for j in range(8):
                accb[pl.ds(base + j * L, L)] = jnp.zeros((L,), jnp.float32)
            return 0

        lax.fori_loop(0, G // (8 * L), zero_block, 0)

        def chunk(k, _):
            off = k * CK
            pltpu.sync_copy(idxs.at[b, pl.ds(off, CK)], idxb)
            pltpu.sync_copy(wks.at[b, :, pl.ds(off, CK)], wkb)
            pltpu.sync_copy(data65.at[b, ch, pl.ds(off, CK)], dbuf)

            def group(g, _):
                g16 = g * L
                d = dbuf[pl.ds(g16, L)]
                i0 = idxb[pl.ds(g16, L)]
                for ci, offc in enumerate((0, 1, 256, 257)):
                    wv = wkb[ci, pl.ds(g16, L)]
                    plsc.addupdate_scatter(accb, [i0 + offc], d * wv)
                return 0

            lax.fori_loop(0, CK // L, group, 0)
            return 0

        lax.fori_loop(0, P // CK, chunk, 0)
        pltpu.sync_copy(accb, acc.at[b, ch])

    for p in range(NPASS // NS):
        do_pass(p * NS + s)
    rem = NPASS % NS
    if rem:
        @pl.when(s < rem)
        def _():
            do_pass((NPASS // NS) * NS + s)


def _splat(pxyz, data65):
    mesh = plsc.VectorSubcoreMesh(core_axis_name="c", subcore_axis_name="s")
    f = functools.partial(
        pl.kernel,
        mesh=mesh,
        compiler_params=pltpu.CompilerParams(needs_layout_passes=False),
        out_type=[
            jax.ShapeDtypeStruct((BS, CH, G), jnp.float32),
            jax.ShapeDtypeStruct((BS, P), jnp.int32),
            jax.ShapeDtypeStruct((BS, 4, P), jnp.float32),
        ],
        scratch_types=[
            pltpu.VMEM((3, PT), jnp.float32),
            pltpu.VMEM((CK,), jnp.int32),
            pltpu.VMEM((4, CK), jnp.float32),
            pltpu.VMEM((CK,), jnp.float32),
            pltpu.VMEM((G,), jnp.float32),
        ],
    )(_splat_body)
    return f(pxyz, data65)


def kernel(xyz, data, h, w):
    bs, p, _ = xyz.shape
    c = data.shape[1]
    x = xyz[..., 0]
    y = xyz[..., 1]
    z = xyz[..., 2]
    viz = (x > -1.0) & (x < 1.0) & (y > -1.0) & (y < 1.0) & (z > 0.0)
    px = (x + 1.0) * 0.5 * (w - 1)
    py = (y + 1.0) * 0.5 * (h - 1)
    iz = viz.astype(jnp.float32) / jnp.maximum(z, 1e-3)
    pxyz = jnp.stack([px, py, iz], axis=1)
    data65 = jnp.concatenate(
        [data, jnp.ones((bs, 1, p), jnp.float32)], axis=1)
    acc, _, _ = _splat(pxyz, data65)
    conf = acc[:, c]
    dmap = (acc[:, :c] / jnp.maximum(conf[:, None], 1e-8)).reshape(bs, c, 256, 256)
    return (dmap, conf.reshape(bs, 1, 256, 256), viz)


# speedup vs baseline: 7.1903x; 2.4762x over previous
import functools

import jax
import jax.numpy as jnp
from jax import lax
from jax.experimental import pallas as pl
from jax.experimental.pallas import tpu as pltpu
from jax.experimental.pallas import tpu_sc as plsc

P = 32768
BS = 4
CH = 65
G = 65536
L = 16
NS = 16
PT = P // NS
CK = 4096
NCHUNK = P // CK
NPASS = 2 * CH


def _splat_body(pxyz, data65, acc, idxs, wks, pbuf, idxb, wkb, dbuf, accb, sems):
    c = lax.axis_index("c")
    s = lax.axis_index("s")

    for bl in range(2):
        b = 2 * c + bl
        base = s * PT
        pltpu.sync_copy(pxyz.at[b, :, pl.ds(base, PT)], pbuf)

        @plsc.parallel_loop(0, PT // L, unroll=4)
        def p1_group(g):
            g16 = g * L
            px = pbuf[0, pl.ds(g16, L)]
            py = pbuf[1, pl.ds(g16, L)]
            iz = pbuf[2, pl.ds(g16, L)]
            pxc = jnp.minimum(jnp.maximum(px, 0.0), 255.0)
            pyc = jnp.minimum(jnp.maximum(py, 0.0), 255.0)
            x0 = jnp.minimum(pxc.astype(jnp.int32), 254)
            y0 = jnp.minimum(pyc.astype(jnp.int32), 254)
            fx = pxc - x0.astype(jnp.float32)
            fy = pyc - y0.astype(jnp.float32)
            gx = 1.0 - fx
            gy = 1.0 - fy
            idxb[0, pl.ds(g16, L)] = y0 * 256 + x0
            wkb[0, 0, pl.ds(g16, L)] = gx * gy * iz
            wkb[0, 1, pl.ds(g16, L)] = fx * gy * iz
            wkb[0, 2, pl.ds(g16, L)] = gx * fy * iz
            wkb[0, 3, pl.ds(g16, L)] = fx * fy * iz

        pltpu.sync_copy(idxb.at[0, pl.ds(0, PT)], idxs.at[b, pl.ds(base, PT)])
        pltpu.sync_copy(wkb.at[0, :, pl.ds(0, PT)], wks.at[b, :, pl.ds(base, PT)])

    plsc.subcore_barrier()

    def start(slot, b, ch, off):
        pltpu.async_copy(idxs.at[b, pl.ds(off, CK)], idxb.at[slot], sems.at[slot, 0])
        pltpu.async_copy(wks.at[b, :, pl.ds(off, CK)], wkb.at[slot], sems.at[slot, 1])
        pltpu.async_copy(data65.at[b, ch, pl.ds(off, CK)], dbuf.at[slot], sems.at[slot, 2])

    def wait(slot, b, ch, off):
        pltpu.make_async_copy(idxs.at[b, pl.ds(off, CK)], idxb.at[slot], sems.at[slot, 0]).wait()
        pltpu.make_async_copy(wks.at[b, :, pl.ds(off, CK)], wkb.at[slot], sems.at[slot, 1]).wait()
        pltpu.make_async_copy(data65.at[b, ch, pl.ds(off, CK)], dbuf.at[slot], sems.at[slot, 2]).wait()

    def do_pass(lin):
        b = 2 * c + lin // CH
        ch = lin % CH
        start(0, b, ch, 0)
        start(1, b, ch, CK)

        @plsc.parallel_loop(0, G // (8 * L), unroll=4)
        def zero_block(i):
            base = i * (8 * L)
            for j in range(8):
                accb[pl.ds(base + j * L, L)] = jnp.zeros((L,), jnp.float32)

        def chunk2(kk, _):
            for par in range(2):
                k = kk * 2 + par
                off = k * CK
                wait(par, b, ch, off)

                @plsc.parallel_loop(0, CK // L, unroll=4)
                def group(g):
                    g16 = g * L
                    d = dbuf[par, pl.ds(g16, L)]
                    i0 = idxb[par, pl.ds(g16, L)]
                    for ci, offc in enumerate((0, 1, 256, 257)):
                        wv = wkb[par, ci, pl.ds(g16, L)]
                        plsc.addupdate_scatter(accb, [i0 + offc], d * wv)

                @pl.when(k + 2 < NCHUNK)
                def _():
                    start(par, b, ch, off + 2 * CK)

            return 0

        lax.fori_loop(0, NCHUNK // 2, chunk2, 0)
        pltpu.sync_copy(accb, acc.at[b, ch])

    for p in range(NPASS // NS):
        do_pass(p * NS + s)
    rem = NPASS % NS
    if rem:
        @pl.when(s < rem)
        def _():
            do_pass((NPASS // NS) * NS + s)


def _splat(pxyz, data65):
    mesh = plsc.VectorSubcoreMesh(core_axis_name="c", subcore_axis_name="s")
    f = functools.partial(
        pl.kernel,
        mesh=mesh,
        compiler_params=pltpu.CompilerParams(needs_layout_passes=False),
        out_type=[
            jax.ShapeDtypeStruct((BS, CH, G), jnp.float32),
            jax.ShapeDtypeStruct((BS, P), jnp.int32),
            jax.ShapeDtypeStruct((BS, 4, P), jnp.float32),
        ],
        scratch_types=[
            pltpu.VMEM((3, PT), jnp.float32),
            pltpu.VMEM((2, CK), jnp.int32),
            pltpu.VMEM((2, 4, CK), jnp.float32),
            pltpu.VMEM((2, CK), jnp.float32),
            pltpu.VMEM((G,), jnp.float32),
            pltpu.SemaphoreType.DMA((2, 3)),
        ],
    )(_splat_body)
    return f(pxyz, data65)


def kernel(xyz, data, h, w):
    bs, p, _ = xyz.shape
    c = data.shape[1]
    x = xyz[..., 0]
    y = xyz[..., 1]
    z = xyz[..., 2]
    viz = (x > -1.0) & (x < 1.0) & (y > -1.0) & (y < 1.0) & (z > 0.0)
    px = (x + 1.0) * 0.5 * (w - 1)
    py = (y + 1.0) * 0.5 * (h - 1)
    iz = viz.astype(jnp.float32) / jnp.maximum(z, 1e-3)
    pxyz = jnp.stack([px, py, iz], axis=1)
    data65 = jnp.concatenate(
        [data, jnp.ones((bs, 1, p), jnp.float32)], axis=1)
    acc, _, _ = _splat(pxyz, data65)
    conf = acc[:, c]
    dmap = (acc[:, :c] / jnp.maximum(conf[:, None], 1e-8)).reshape(bs, c, 256, 256)
    return (dmap, conf.reshape(bs, 1, 256, 256), viz)
